# baseline (speedup 1.0000x reference)
import functools

import jax
import jax.numpy as jnp
from jax import lax
from jax.experimental import pallas as pl
from jax.experimental.pallas import tpu as pltpu
from jax.experimental.pallas import tpu_sc as plsc

VOCAB = 1000000
D = 64
N_TOK = 20
B = 1024
S = 200
OUT_S = N_TOK + S

NC = 2
NS = 16
NW = NC * NS

NJF = 7812
PACK_ROWS = 500000

BPW = B // NW
CH = 100
NCH = S // CH
G = 4
NG = BPW // G


def _transpose_body(wteT_hbm, tail_hbm, out_hbm, chunk_a, chunk_b, obuf_a,
                    obuf_b, tbuf, fsem_a, fsem_b, wsem_a, wsem_b):
    wid = lax.axis_index("s") * NC + lax.axis_index("c")
    iota16 = lax.iota(jnp.int32, 16)

    chunks = ((chunk_a, fsem_a), (chunk_b, fsem_b))
    obufs = ((obuf_a, wsem_a), (obuf_b, wsem_b))

    def fetch(j, chunk, fsem):
        off = pl.multiple_of(j * 128, 128)
        pltpu.async_copy(wteT_hbm.at[:, pl.ds(off, 128)], chunk, fsem)

    def wait_fetch(j, chunk, fsem):
        pltpu.make_async_copy(
            wteT_hbm.at[:, pl.ds(0, 128)], chunk, fsem).wait()

    def transpose(chunk, obuf):
        rowv = [iota16 + (16 * q) for q in range(4)]
        for pr in range(64):
            for c in range(8):
                col = jnp.full((16,), 2 * pr + (c // 4), jnp.int32)
                v = plsc.load_gather(chunk, [rowv[c % 4], col])
                obuf[pr, pl.ds(16 * c, 16)] = v

    def n_body(n, carry):
        j0 = wid + 32 * (2 * n)
        j1 = wid + 32 * (2 * n + 1)
        for (j, (chunk, fsem), (obuf, wsem)) in ((j0, chunks[0], obufs[0]),
                                                 (j1, chunks[1], obufs[1])):
            @pl.when(j < NJF)
            def _():
                wait_fetch(j, chunk, fsem)
            @pl.when(jnp.logical_and(j >= 64, j < NJF))
            def _():
                pltpu.make_async_copy(
                    obuf, out_hbm.at[pl.ds(0, 64)], wsem).wait()
            @pl.when(j < NJF)
            def _():
                transpose(chunk, obuf)
                pltpu.async_copy(
                    obuf, out_hbm.at[pl.ds(pl.multiple_of(j * 64, 64), 64)],
                    wsem)
            jn = j + 64
            @pl.when(jn < NJF)
            def _():
                fetch(jn, chunk, fsem)
        return carry

    @pl.when(wid == 0)
    def _():
        pltpu.sync_copy(tail_hbm, tbuf)
        pltpu.sync_copy(tbuf, out_hbm.at[pl.ds(PACK_ROWS - 32, 32)])

    fetch(wid, chunk_a, fsem_a)
    fetch(wid + 32, chunk_b, fsem_b)
    lax.fori_loop(0, 123, n_body, 0)
    for (obuf, wsem) in obufs:
        pltpu.make_async_copy(obuf, out_hbm.at[pl.ds(0, 64)], wsem).wait()


def _gather_body(tokens_hbm, wte_hbm, learned_hbm, out_hbm,
                 idx_v, buf_a, buf_b, gsem_a, gsem_b, wsem_a, wsem_b):
    wid = lax.axis_index("s") * NC + lax.axis_index("c")
    base = wid * BPW

    pltpu.sync_copy(tokens_hbm.at[pl.ds(base, BPW)], idx_v)
    for buf in (buf_a, buf_b):
        for k in range(G):
            pltpu.sync_copy(learned_hbm, buf.at[k, pl.ds(0, N_TOK)])

    bufs = ((buf_a, gsem_a, wsem_a), (buf_b, gsem_b, wsem_b))

    def issue_gathers(g, buf, gsem):
        descs = []
        for k in range(G):
            i = g * G + k
            for j in range(NCH):
                descs.append(pltpu.async_copy(
                    wte_hbm.at[idx_v.at[i, j]],
                    buf.at[k, pl.ds(N_TOK + j * CH, CH)],
                    gsem,
                ))
        return descs

    pending_g = {0: issue_gathers(0, buf_a, gsem_a), 1: None}
    pending_w = {0: None, 1: None}

    for g in range(NG):
        p = g % 2
        buf, gsem, wsem = bufs[p]
        for dsc in pending_g[p]:
            dsc.wait()
        pending_w[p] = pltpu.async_copy(
            buf, out_hbm.at[pl.ds(base + g * G, G)], wsem)
        if g + 1 < NG:
            q = 1 - p
            if pending_w[q] is not None:
                pending_w[q].wait()
                pending_w[q] = None
            pending_g[q] = issue_gathers(g + 1, bufs[q][0], bufs[q][1])

    for p in (0, 1):
        if pending_w[p] is not None:
            pending_w[p].wait()


@functools.partial(jax.jit)
def kernel(tokens, wte_weight, learned_embedding):
    mesh = plsc.VectorSubcoreMesh(core_axis_name="c", subcore_axis_name="s")

    kt = pl.kernel(
        _transpose_body,
        mesh=mesh,
        out_type=jax.ShapeDtypeStruct((PACK_ROWS, 128), jnp.float32),
        scratch_types=[
            pltpu.VMEM((D, 128), jnp.float32),
            pltpu.VMEM((D, 128), jnp.float32),
            pltpu.VMEM((D, 128), jnp.float32),
            pltpu.VMEM((D, 128), jnp.float32),
            pltpu.VMEM((32, 128), jnp.float32),
            pltpu.SemaphoreType.DMA,
            pltpu.SemaphoreType.DMA,
            pltpu.SemaphoreType.DMA,
            pltpu.SemaphoreType.DMA,
        ],
        compiler_params=pltpu.CompilerParams(
            use_tc_tiling_on_sc=True, needs_layout_passes=False),
    )
    wte_lin = kt(wte_weight.T, wte_weight[VOCAB - 64:].reshape(32, 128)).reshape(VOCAB, D)

    tokens3 = tokens.reshape(B, NCH, CH).astype(jnp.int32)
    kg = pl.kernel(
        _gather_body,
        mesh=mesh,
        out_type=jax.ShapeDtypeStruct((B, OUT_S, D), jnp.float32),
        scratch_types=[
            pltpu.VMEM((BPW, NCH, CH), jnp.int32),
            pltpu.VMEM((G, OUT_S, D), jnp.float32),
            pltpu.VMEM((G, OUT_S, D), jnp.float32),
            pltpu.SemaphoreType.DMA,
            pltpu.SemaphoreType.DMA,
            pltpu.SemaphoreType.DMA,
            pltpu.SemaphoreType.DMA,
        ],
        compiler_params=pltpu.CompilerParams(use_tc_tiling_on_sc=False),
    )
    return kg(tokens3, wte_lin, learned_embedding)


# speedup vs baseline: 2.4760x; 2.4760x over previous
import functools

import jax
import jax.numpy as jnp
from jax import lax
from jax.experimental import pallas as pl
from jax.experimental.pallas import tpu as pltpu
from jax.experimental.pallas import tpu_sc as plsc

VOCAB = 1000000
D = 64
N_TOK = 20
B = 1024
S = 200
OUT_S = N_TOK + S

NC = 2
NS = 16
NW = NC * NS
BPW = B // NW
CH = 100
NCH = S // CH
G = 4
NG = BPW // G


def _body(tokens_hbm, wte_hbm, learned_hbm, out_hbm,
          idx_v, buf_a, buf_b, gsem_a, gsem_b, wsem_a, wsem_b):
    wid = lax.axis_index("s") * NC + lax.axis_index("c")
    base = wid * BPW

    pltpu.sync_copy(tokens_hbm.at[pl.ds(base, BPW)], idx_v)

    for buf in (buf_a, buf_b):
        for k in range(G):
            pltpu.sync_copy(learned_hbm, buf.at[k, pl.ds(0, N_TOK)])

    bufs = ((buf_a, gsem_a, wsem_a), (buf_b, gsem_b, wsem_b))

    def issue_gathers(g, buf, gsem):
        descs = []
        for k in range(G):
            i = g * G + k
            for j in range(NCH):
                descs.append(pltpu.async_copy(
                    wte_hbm.at[idx_v.at[i, j]],
                    buf.at[k, pl.ds(N_TOK + j * CH, CH)],
                    gsem,
                ))
        return descs

    pending_g = {0: issue_gathers(0, buf_a, gsem_a), 1: None}
    pending_w = {0: None, 1: None}

    for g in range(NG):
        p = g % 2
        buf, gsem, wsem = bufs[p]
        for dsc in pending_g[p]:
            dsc.wait()
        pending_w[p] = pltpu.async_copy(
            buf, out_hbm.at[pl.ds(base + g * G, G)], wsem)
        if g + 1 < NG:
            q = 1 - p
            if pending_w[q] is not None:
                pending_w[q].wait()
                pending_w[q] = None
            pending_g[q] = issue_gathers(g + 1, bufs[q][0], bufs[q][1])

    for p in (0, 1):
        if pending_w[p] is not None:
            pending_w[p].wait()


@functools.partial(jax.jit)
def kernel(tokens, wte_weight, learned_embedding):
    tokens3 = tokens.reshape(B, NCH, CH).astype(jnp.int32)
    mesh = plsc.VectorSubcoreMesh(core_axis_name="c", subcore_axis_name="s")
    k = pl.kernel(
        _body,
        mesh=mesh,
        out_type=jax.ShapeDtypeStruct((B, OUT_S, D), jnp.float32),
        scratch_types=[
            pltpu.VMEM((BPW, NCH, CH), jnp.int32),
            pltpu.VMEM((G, OUT_S, D), jnp.float32),
            pltpu.VMEM((G, OUT_S, D), jnp.float32),
            pltpu.SemaphoreType.DMA,
            pltpu.SemaphoreType.DMA,
            pltpu.SemaphoreType.DMA,
            pltpu.SemaphoreType.DMA,
        ],
        compiler_params=pltpu.CompilerParams(use_tc_tiling_on_sc=False),
    )
    return k(tokens3, wte_weight, learned_embedding)
